# baseline (speedup 1.0000x reference)
import functools

import jax
import jax.numpy as jnp
from jax.experimental import pallas as pl
from jax.experimental.pallas import tpu as pltpu

_K = 16
_GROUPS = 4
_EPS = 1e-5
_F32 = jnp.float32
_HI = jax.lax.Precision.HIGHEST


def _mm(a, b, ca, cb, precision=None):
    return jax.lax.dot_general(a, b, (((ca,), (cb,)), ((), ())),
                               preferred_element_type=_F32,
                               precision=precision)


def _fps_kernel(coor_ref, idx_ref, coorq_ref, dist_sc, far_sc, *, M):
    B = coor_ref.shape[0]
    N = coor_ref.shape[2]
    x0 = coor_ref[:, 0, :]
    x1 = coor_ref[:, 1, :]
    x2 = coor_ref[:, 2, :]
    iota_n = jax.lax.broadcasted_iota(jnp.int32, (B, N), 1)
    iota_m = jax.lax.broadcasted_iota(jnp.int32, (B, M), 1)

    dist_sc[...] = jnp.full((B, N), 1e10, _F32)
    far_sc[...] = jnp.zeros((B, 1), jnp.int32)
    idx_ref[...] = jnp.zeros((B, M), jnp.int32)
    coorq_ref[...] = jnp.zeros((B, 3, M), _F32)

    def body(i, carry):
        far = far_sc[...]
        sel = iota_m == i
        idx_ref[...] = jnp.where(sel, far, idx_ref[...])
        oh = (iota_n == far).astype(_F32)
        c0 = jnp.sum(x0 * oh, axis=1, keepdims=True)
        c1 = jnp.sum(x1 * oh, axis=1, keepdims=True)
        c2 = jnp.sum(x2 * oh, axis=1, keepdims=True)
        coorq_ref[:, 0, :] = jnp.where(sel, c0, coorq_ref[:, 0, :])
        coorq_ref[:, 1, :] = jnp.where(sel, c1, coorq_ref[:, 1, :])
        coorq_ref[:, 2, :] = jnp.where(sel, c2, coorq_ref[:, 2, :])
        d0 = x0 - c0
        d1 = x1 - c1
        d2 = x2 - c2
        d = d0 * d0 + d1 * d1 + d2 * d2
        dist = jnp.minimum(dist_sc[...], d)
        dist_sc[...] = dist
        mx = jnp.max(dist, axis=1, keepdims=True)
        far_sc[...] = jnp.min(jnp.where(dist == mx, iota_n, N), axis=1,
                              keepdims=True)
        return carry

    jax.lax.fori_loop(0, M, body, 0)


def _fps(coor, M):
    B = coor.shape[0]
    N = coor.shape[2]
    G = 2
    Bg = B // G
    return pl.pallas_call(
        functools.partial(_fps_kernel, M=M),
        grid=(G,),
        in_specs=[pl.BlockSpec((Bg, 3, N), lambda g: (g, 0, 0))],
        out_specs=[pl.BlockSpec((Bg, M), lambda g: (g, 0)),
                   pl.BlockSpec((Bg, 3, M), lambda g: (g, 0, 0))],
        out_shape=[jax.ShapeDtypeStruct((B, M), jnp.int32),
                   jax.ShapeDtypeStruct((B, 3, M), _F32)],
        scratch_shapes=[pltpu.VMEM((Bg, N), _F32),
                        pltpu.VMEM((Bg, 1), jnp.int32)],
        compiler_params=pltpu.CompilerParams(
            dimension_semantics=("parallel",)),
    )(coor)


def _group_mat(O):
    Og = O // _GROUPS
    a = jax.lax.broadcasted_iota(jnp.int32, (O, _GROUPS), 0) // Og
    b = jax.lax.broadcasted_iota(jnp.int32, (O, _GROUPS), 1)
    return (a == b).astype(_F32)


def _dist(cq, ck):
    qk = _mm(cq, ck, 0, 0)
    qsq = jnp.sum(cq * cq, axis=0, keepdims=True)
    ksq = jnp.sum(ck * ck, axis=0, keepdims=True)
    qcol = _mm(qsq, jnp.ones((1, 1), _F32), 0, 0, _HI)
    d = -2.0 * qk
    d = d + qcol
    return d + ksq


def _topk_conv(d0, fkT, fqT, W, nt, d_sc, mx_sc, s1_sc, s2_sc,
               Mx_ref, stats_ref):
    TQ, Nk = d0.shape
    O = W.shape[0]
    d_sc[...] = d0
    mx_sc[...] = jnp.full(mx_sc.shape, -1e30, _F32)
    s1_sc[...] = jnp.zeros(s1_sc.shape, _F32)
    s2_sc[...] = jnp.zeros(s2_sc.shape, _F32)
    iota = jax.lax.broadcasted_iota(jnp.int32, (TQ, Nk), 1)

    def body(j, carry):
        dd = d_sc[...]
        m = jnp.min(dd, axis=1, keepdims=True)
        istar = jnp.min(jnp.where(dd == m, iota, Nk), axis=1, keepdims=True)
        oh = (iota == istar).astype(_F32)
        d_sc[...] = dd + oh * 1e30
        fkj = _mm(fkT, oh, 1, 1, _HI)
        edge = jnp.concatenate([fkj - fqT, fqT], axis=0)
        cv = _mm(W, edge, 1, 0)
        mx_sc[...] = jnp.maximum(mx_sc[...], cv)
        s1_sc[...] = s1_sc[...] + cv
        s2_sc[...] = s2_sc[...] + cv * cv
        return carry

    jax.lax.fori_loop(0, _K, body, 0)

    Mx_ref[0] = mx_sc[...]
    Mg = _group_mat(O)
    t1 = jnp.sum(s1_sc[...], axis=1, keepdims=True)
    t2 = jnp.sum(s2_sc[...], axis=1, keepdims=True)
    gs1 = _mm(Mg, t1, 0, 0, _HI)
    gs2 = _mm(Mg, t2, 0, 0, _HI)
    st = jnp.concatenate([gs1, gs2], axis=1)

    @pl.when(nt == 0)
    def _():
        stats_ref[0] = st

    @pl.when(nt != 0)
    def _():
        stats_ref[0] = stats_ref[0] + st


def _stage1_kernel(cq_ref, ck_ref, Win_ref, b_ref, W_ref,
                   Mx_ref, stats_ref, d_sc, mx_sc, s1_sc, s2_sc):
    cq = cq_ref[0]
    ck = ck_ref[0]
    fkT = _mm(Win_ref[...], ck, 1, 0) + b_ref[...]
    fqT = _mm(Win_ref[...], cq, 1, 0) + b_ref[...]
    _topk_conv(_dist(cq, ck), fkT, fqT, W_ref[...], pl.program_id(1),
               d_sc, mx_sc, s1_sc, s2_sc, Mx_ref, stats_ref)


def _stage_self_kernel(ck_ref, fk_ref, W_ref,
                       Mx_ref, stats_ref, d_sc, mx_sc, s1_sc, s2_sc):
    ck = ck_ref[0]
    fk = fk_ref[0]
    _topk_conv(_dist(ck, ck), fk, fk, W_ref[...], 0,
               d_sc, mx_sc, s1_sc, s2_sc, Mx_ref, stats_ref)


def _stage_gather_kernel(cq_ref, ck_ref, fk_ref, idx_ref, W_ref,
                         Mx_ref, stats_ref, d_sc, mx_sc, s1_sc, s2_sc):
    cq = cq_ref[0]
    ck = ck_ref[0]
    fk = fk_ref[0]
    Nk = fk.shape[1]
    Nq = cq.shape[1]
    idx = idx_ref[0]
    PT = (jax.lax.broadcasted_iota(jnp.int32, (Nk, Nq), 0) == idx).astype(_F32)
    fqT = _mm(fk, PT, 1, 0, _HI)
    _topk_conv(_dist(cq, ck), fk, fqT, W_ref[...], 0,
               d_sc, mx_sc, s1_sc, s2_sc, Mx_ref, stats_ref)


def _finalize_kernel(Mx_ref, stats_ref, gamma_ref, beta_ref, out_ref, *,
                     count):
    st = stats_ref[0]
    mean_g = st[:, 0:1] / count
    var_g = st[:, 1:2] / count - mean_g * mean_g
    rstd_g = jax.lax.rsqrt(var_g + _EPS)
    O = out_ref.shape[1]
    Mg = _group_mat(O)
    mean_c = _mm(Mg, mean_g, 1, 0, _HI)
    rstd_c = _mm(Mg, rstd_g, 1, 0, _HI)
    y = (Mx_ref[0] - mean_c) * (rstd_c * gamma_ref[...]) + beta_ref[...]
    out_ref[0] = jnp.where(y > 0, y, 0.2 * y)


def _stage_scratch(TQ, Nk, O):
    return [pltpu.VMEM((TQ, Nk), _F32), pltpu.VMEM((O, TQ), _F32),
            pltpu.VMEM((O, TQ), _F32), pltpu.VMEM((O, TQ), _F32)]


def _full(shape):
    nd = len(shape)
    return pl.BlockSpec(shape, lambda *idx: (0,) * nd)


def _finalize(Mx, stats, gamma, beta, count):
    B, O, Nq = Mx.shape
    return pl.pallas_call(
        functools.partial(_finalize_kernel, count=count),
        grid=(B,),
        in_specs=[pl.BlockSpec((1, O, Nq), lambda b: (b, 0, 0)),
                  pl.BlockSpec((1, _GROUPS, 2), lambda b: (b, 0, 0)),
                  _full(gamma.shape), _full(beta.shape)],
        out_specs=pl.BlockSpec((1, O, Nq), lambda b: (b, 0, 0)),
        out_shape=jax.ShapeDtypeStruct((B, O, Nq), _F32),
        compiler_params=pltpu.CompilerParams(
            dimension_semantics=("parallel",)),
    )(Mx, stats, gamma, beta)


def _run_stage1(x, W_in, b_in, W1, gamma, beta, TQ=512):
    B, _, N = x.shape
    O = W1.shape[0]
    NT = N // TQ
    Mx, stats = pl.pallas_call(
        _stage1_kernel,
        grid=(B, NT),
        in_specs=[pl.BlockSpec((1, 3, TQ), lambda b, t: (b, 0, t)),
                  pl.BlockSpec((1, 3, N), lambda b, t: (b, 0, 0)),
                  _full(W_in.shape), _full(b_in.shape), _full(W1.shape)],
        out_specs=[pl.BlockSpec((1, O, TQ), lambda b, t: (b, 0, t)),
                   pl.BlockSpec((1, _GROUPS, 2), lambda b, t: (b, 0, 0))],
        out_shape=[jax.ShapeDtypeStruct((B, O, N), _F32),
                   jax.ShapeDtypeStruct((B, _GROUPS, 2), _F32)],
        scratch_shapes=_stage_scratch(TQ, N, O),
        compiler_params=pltpu.CompilerParams(
            dimension_semantics=("parallel", "arbitrary")),
    )(x, x, W_in, b_in, W1)
    return _finalize(Mx, stats, gamma, beta, float(N * _K * (O // _GROUPS)))


def _run_stage_self(coor, f, W, gamma, beta):
    B, _, N = coor.shape
    C = f.shape[1]
    O = W.shape[0]
    Mx, stats = pl.pallas_call(
        _stage_self_kernel,
        grid=(B,),
        in_specs=[pl.BlockSpec((1, 3, N), lambda b: (b, 0, 0)),
                  pl.BlockSpec((1, C, N), lambda b: (b, 0, 0)),
                  _full(W.shape)],
        out_specs=[pl.BlockSpec((1, O, N), lambda b: (b, 0, 0)),
                   pl.BlockSpec((1, _GROUPS, 2), lambda b: (b, 0, 0))],
        out_shape=[jax.ShapeDtypeStruct((B, O, N), _F32),
                   jax.ShapeDtypeStruct((B, _GROUPS, 2), _F32)],
        scratch_shapes=_stage_scratch(N, N, O),
        compiler_params=pltpu.CompilerParams(
            dimension_semantics=("parallel",)),
    )(coor, f, W)
    return _finalize(Mx, stats, gamma, beta, float(N * _K * (O // _GROUPS)))


def _run_stage_gather(coor_q, coor_k, f_k, idx, W, gamma, beta):
    B, _, Nq = coor_q.shape
    Nk = coor_k.shape[2]
    C = f_k.shape[1]
    O = W.shape[0]
    idx3 = idx.reshape(B, 1, Nq)
    Mx, stats = pl.pallas_call(
        _stage_gather_kernel,
        grid=(B,),
        in_specs=[pl.BlockSpec((1, 3, Nq), lambda b: (b, 0, 0)),
                  pl.BlockSpec((1, 3, Nk), lambda b: (b, 0, 0)),
                  pl.BlockSpec((1, C, Nk), lambda b: (b, 0, 0)),
                  pl.BlockSpec((1, 1, Nq), lambda b: (b, 0, 0)),
                  _full(W.shape)],
        out_specs=[pl.BlockSpec((1, O, Nq), lambda b: (b, 0, 0)),
                   pl.BlockSpec((1, _GROUPS, 2), lambda b: (b, 0, 0))],
        out_shape=[jax.ShapeDtypeStruct((B, O, Nq), _F32),
                   jax.ShapeDtypeStruct((B, _GROUPS, 2), _F32)],
        scratch_shapes=_stage_scratch(Nq, Nk, O),
        compiler_params=pltpu.CompilerParams(
            dimension_semantics=("parallel",)),
    )(coor_q, coor_k, f_k, idx3, W)
    return _finalize(Mx, stats, gamma, beta, float(Nq * _K * (O // _GROUPS)))


def kernel(x, W_in, b_in, W1, g1, bt1, W2, g2, bt2, W3, g3, bt3, W4, g4, bt4):
    bcol = b_in.reshape(-1, 1)
    col = lambda v: v.reshape(-1, 1)

    f1 = _run_stage1(x, W_in, bcol, W1, col(g1), col(bt1))
    idx1, coorq1 = _fps(x, 512)
    f2 = _run_stage_gather(coorq1, x, f1, idx1, W2, col(g2), col(bt2))
    f3 = _run_stage_self(coorq1, f2, W3, col(g3), col(bt3))
    idx2, coorq2 = _fps(coorq1, 128)
    f4 = _run_stage_gather(coorq2, coorq1, f3, idx2, W4, col(g4), col(bt4))
    return coorq2, f4


# speedup vs baseline: 1.0350x; 1.0350x over previous
import functools

import jax
import jax.numpy as jnp
from jax.experimental import pallas as pl
from jax.experimental.pallas import tpu as pltpu

_K = 16
_GROUPS = 4
_EPS = 1e-5
_F32 = jnp.float32
_HI = jax.lax.Precision.HIGHEST


def _mm(a, b, ca, cb, precision=None):
    return jax.lax.dot_general(a, b, (((ca,), (cb,)), ((), ())),
                               preferred_element_type=_F32,
                               precision=precision)


def _fps_kernel(coor_ref, idx_ref, coorq_ref, dist_sc, far_sc, *, M):
    B = coor_ref.shape[0]
    N = coor_ref.shape[2]
    x0 = coor_ref[:, 0, :]
    x1 = coor_ref[:, 1, :]
    x2 = coor_ref[:, 2, :]
    iota_n = jax.lax.broadcasted_iota(jnp.int32, (B, N), 1)
    iota_m = jax.lax.broadcasted_iota(jnp.int32, (B, M), 1)

    dist_sc[...] = jnp.full((B, N), 1e10, _F32)
    far_sc[...] = jnp.zeros((B, 1), jnp.int32)
    idx_ref[...] = jnp.zeros((B, M), jnp.int32)
    coorq_ref[...] = jnp.zeros((B, 3, M), _F32)

    def body(i, carry):
        far = far_sc[...]
        sel = iota_m == i
        idx_ref[...] = jnp.where(sel, far, idx_ref[...])
        oh = (iota_n == far).astype(_F32)
        c0 = jnp.sum(x0 * oh, axis=1, keepdims=True)
        c1 = jnp.sum(x1 * oh, axis=1, keepdims=True)
        c2 = jnp.sum(x2 * oh, axis=1, keepdims=True)
        coorq_ref[:, 0, :] = jnp.where(sel, c0, coorq_ref[:, 0, :])
        coorq_ref[:, 1, :] = jnp.where(sel, c1, coorq_ref[:, 1, :])
        coorq_ref[:, 2, :] = jnp.where(sel, c2, coorq_ref[:, 2, :])
        d0 = x0 - c0
        d1 = x1 - c1
        d2 = x2 - c2
        d = d0 * d0 + d1 * d1 + d2 * d2
        dist = jnp.minimum(dist_sc[...], d)
        dist_sc[...] = dist
        mx = jnp.max(dist, axis=1, keepdims=True)
        far_sc[...] = jnp.min(jnp.where(dist == mx, iota_n, N), axis=1,
                              keepdims=True)
        return carry

    jax.lax.fori_loop(0, M, body, 0)


def _fps(coor, M):
    B = coor.shape[0]
    N = coor.shape[2]
    return pl.pallas_call(
        functools.partial(_fps_kernel, M=M),
        out_shape=[jax.ShapeDtypeStruct((B, M), jnp.int32),
                   jax.ShapeDtypeStruct((B, 3, M), _F32)],
        scratch_shapes=[pltpu.VMEM((B, N), _F32),
                        pltpu.VMEM((B, 1), jnp.int32)],
    )(coor)


def _group_mat(O):
    Og = O // _GROUPS
    a = jax.lax.broadcasted_iota(jnp.int32, (O, _GROUPS), 0) // Og
    b = jax.lax.broadcasted_iota(jnp.int32, (O, _GROUPS), 1)
    return (a == b).astype(_F32)


def _dist(cq, ck):
    qk = _mm(cq, ck, 0, 0)
    qsq = jnp.sum(cq * cq, axis=0, keepdims=True)
    ksq = jnp.sum(ck * ck, axis=0, keepdims=True)
    qcol = _mm(qsq, jnp.ones((1, 1), _F32), 0, 0, _HI)
    d = -2.0 * qk
    d = d + qcol
    return d + ksq


def _topk_conv(d0, fkT, fqT, W, nt, d_sc, mx_sc, s1_sc, s2_sc,
               Mx_ref, stats_ref):
    TQ, Nk = d0.shape
    O = W.shape[0]
    d_sc[...] = d0
    mx_sc[...] = jnp.full(mx_sc.shape, -1e30, _F32)
    s1_sc[...] = jnp.zeros(s1_sc.shape, _F32)
    s2_sc[...] = jnp.zeros(s2_sc.shape, _F32)
    iota = jax.lax.broadcasted_iota(jnp.int32, (TQ, Nk), 1)

    def body(j, carry):
        dd = d_sc[...]
        istar = jnp.argmin(dd, axis=1, keepdims=True)
        sel = iota == istar
        d_sc[...] = jnp.where(sel, 1e30, dd)
        oh = sel.astype(_F32)
        fkj = _mm(fkT, oh, 1, 1, _HI)
        edge = jnp.concatenate([fkj - fqT, fqT], axis=0)
        cv = _mm(W, edge, 1, 0)
        mx_sc[...] = jnp.maximum(mx_sc[...], cv)
        s1_sc[...] = s1_sc[...] + cv
        s2_sc[...] = s2_sc[...] + cv * cv
        return carry

    jax.lax.fori_loop(0, _K, body, 0)

    Mx_ref[0] = mx_sc[...]
    Mg = _group_mat(O)
    t1 = jnp.sum(s1_sc[...], axis=1, keepdims=True)
    t2 = jnp.sum(s2_sc[...], axis=1, keepdims=True)
    gs1 = _mm(Mg, t1, 0, 0, _HI)
    gs2 = _mm(Mg, t2, 0, 0, _HI)
    st = jnp.concatenate([gs1, gs2], axis=1)

    @pl.when(nt == 0)
    def _():
        stats_ref[0] = st

    @pl.when(nt != 0)
    def _():
        stats_ref[0] = stats_ref[0] + st


def _stage1_kernel(cq_ref, ck_ref, Win_ref, b_ref, W_ref,
                   Mx_ref, stats_ref, d_sc, mx_sc, s1_sc, s2_sc):
    cq = cq_ref[0]
    ck = ck_ref[0]
    fkT = _mm(Win_ref[...], ck, 1, 0) + b_ref[...]
    fqT = _mm(Win_ref[...], cq, 1, 0) + b_ref[...]
    _topk_conv(_dist(cq, ck), fkT, fqT, W_ref[...], pl.program_id(1),
               d_sc, mx_sc, s1_sc, s2_sc, Mx_ref, stats_ref)


def _stage_self_kernel(ck_ref, fk_ref, W_ref,
                       Mx_ref, stats_ref, d_sc, mx_sc, s1_sc, s2_sc):
    ck = ck_ref[0]
    fk = fk_ref[0]
    _topk_conv(_dist(ck, ck), fk, fk, W_ref[...], 0,
               d_sc, mx_sc, s1_sc, s2_sc, Mx_ref, stats_ref)


def _stage_gather_kernel(cq_ref, ck_ref, fk_ref, idx_ref, W_ref,
                         Mx_ref, stats_ref, d_sc, mx_sc, s1_sc, s2_sc):
    cq = cq_ref[0]
    ck = ck_ref[0]
    fk = fk_ref[0]
    Nk = fk.shape[1]
    Nq = cq.shape[1]
    idx = idx_ref[0]
    PT = (jax.lax.broadcasted_iota(jnp.int32, (Nk, Nq), 0) == idx).astype(_F32)
    fqT = _mm(fk, PT, 1, 0, _HI)
    _topk_conv(_dist(cq, ck), fk, fqT, W_ref[...], 0,
               d_sc, mx_sc, s1_sc, s2_sc, Mx_ref, stats_ref)


def _finalize_kernel(Mx_ref, stats_ref, gamma_ref, beta_ref, out_ref, *,
                     count):
    st = stats_ref[0]
    mean_g = st[:, 0:1] / count
    var_g = st[:, 1:2] / count - mean_g * mean_g
    rstd_g = jax.lax.rsqrt(var_g + _EPS)
    O = out_ref.shape[1]
    Mg = _group_mat(O)
    mean_c = _mm(Mg, mean_g, 1, 0, _HI)
    rstd_c = _mm(Mg, rstd_g, 1, 0, _HI)
    y = (Mx_ref[0] - mean_c) * (rstd_c * gamma_ref[...]) + beta_ref[...]
    out_ref[0] = jnp.where(y > 0, y, 0.2 * y)


def _stage_scratch(TQ, Nk, O):
    return [pltpu.VMEM((TQ, Nk), _F32), pltpu.VMEM((O, TQ), _F32),
            pltpu.VMEM((O, TQ), _F32), pltpu.VMEM((O, TQ), _F32)]


def _full(shape):
    nd = len(shape)
    return pl.BlockSpec(shape, lambda *idx: (0,) * nd)


def _finalize(Mx, stats, gamma, beta, count):
    B, O, Nq = Mx.shape
    return pl.pallas_call(
        functools.partial(_finalize_kernel, count=count),
        grid=(B,),
        in_specs=[pl.BlockSpec((1, O, Nq), lambda b: (b, 0, 0)),
                  pl.BlockSpec((1, _GROUPS, 2), lambda b: (b, 0, 0)),
                  _full(gamma.shape), _full(beta.shape)],
        out_specs=pl.BlockSpec((1, O, Nq), lambda b: (b, 0, 0)),
        out_shape=jax.ShapeDtypeStruct((B, O, Nq), _F32),
    )(Mx, stats, gamma, beta)


def _run_stage1(x, W_in, b_in, W1, gamma, beta, TQ=512):
    B, _, N = x.shape
    O = W1.shape[0]
    NT = N // TQ
    Mx, stats = pl.pallas_call(
        _stage1_kernel,
        grid=(B, NT),
        in_specs=[pl.BlockSpec((1, 3, TQ), lambda b, t: (b, 0, t)),
                  pl.BlockSpec((1, 3, N), lambda b, t: (b, 0, 0)),
                  _full(W_in.shape), _full(b_in.shape), _full(W1.shape)],
        out_specs=[pl.BlockSpec((1, O, TQ), lambda b, t: (b, 0, t)),
                   pl.BlockSpec((1, _GROUPS, 2), lambda b, t: (b, 0, 0))],
        out_shape=[jax.ShapeDtypeStruct((B, O, N), _F32),
                   jax.ShapeDtypeStruct((B, _GROUPS, 2), _F32)],
        scratch_shapes=_stage_scratch(TQ, N, O),
    )(x, x, W_in, b_in, W1)
    return _finalize(Mx, stats, gamma, beta, float(N * _K * (O // _GROUPS)))


def _run_stage_self(coor, f, W, gamma, beta):
    B, _, N = coor.shape
    C = f.shape[1]
    O = W.shape[0]
    Mx, stats = pl.pallas_call(
        _stage_self_kernel,
        grid=(B,),
        in_specs=[pl.BlockSpec((1, 3, N), lambda b: (b, 0, 0)),
                  pl.BlockSpec((1, C, N), lambda b: (b, 0, 0)),
                  _full(W.shape)],
        out_specs=[pl.BlockSpec((1, O, N), lambda b: (b, 0, 0)),
                   pl.BlockSpec((1, _GROUPS, 2), lambda b: (b, 0, 0))],
        out_shape=[jax.ShapeDtypeStruct((B, O, N), _F32),
                   jax.ShapeDtypeStruct((B, _GROUPS, 2), _F32)],
        scratch_shapes=_stage_scratch(N, N, O),
    )(coor, f, W)
    return _finalize(Mx, stats, gamma, beta, float(N * _K * (O // _GROUPS)))


def _run_stage_gather(coor_q, coor_k, f_k, idx, W, gamma, beta):
    B, _, Nq = coor_q.shape
    Nk = coor_k.shape[2]
    C = f_k.shape[1]
    O = W.shape[0]
    idx3 = idx.reshape(B, 1, Nq)
    Mx, stats = pl.pallas_call(
        _stage_gather_kernel,
        grid=(B,),
        in_specs=[pl.BlockSpec((1, 3, Nq), lambda b: (b, 0, 0)),
                  pl.BlockSpec((1, 3, Nk), lambda b: (b, 0, 0)),
                  pl.BlockSpec((1, C, Nk), lambda b: (b, 0, 0)),
                  pl.BlockSpec((1, 1, Nq), lambda b: (b, 0, 0)),
                  _full(W.shape)],
        out_specs=[pl.BlockSpec((1, O, Nq), lambda b: (b, 0, 0)),
                   pl.BlockSpec((1, _GROUPS, 2), lambda b: (b, 0, 0))],
        out_shape=[jax.ShapeDtypeStruct((B, O, Nq), _F32),
                   jax.ShapeDtypeStruct((B, _GROUPS, 2), _F32)],
        scratch_shapes=_stage_scratch(Nq, Nk, O),
    )(coor_q, coor_k, f_k, idx3, W)
    return _finalize(Mx, stats, gamma, beta, float(Nq * _K * (O // _GROUPS)))


def kernel(x, W_in, b_in, W1, g1, bt1, W2, g2, bt2, W3, g3, bt3, W4, g4, bt4):
    bcol = b_in.reshape(-1, 1)
    col = lambda v: v.reshape(-1, 1)

    f1 = _run_stage1(x, W_in, bcol, W1, col(g1), col(bt1))
    idx1, coorq1 = _fps(x, 512)
    f2 = _run_stage_gather(coorq1, x, f1, idx1, W2, col(g2), col(bt2))
    f3 = _run_stage_self(coorq1, f2, W3, col(g3), col(bt3))
    idx2, coorq2 = _fps(coorq1, 128)
    f4 = _run_stage_gather(coorq2, coorq1, f3, idx2, W4, col(g4), col(bt4))
    return coorq2, f4


# speedup vs baseline: 1.4736x; 1.4238x over previous
import functools

import jax
import jax.numpy as jnp
from jax.experimental import pallas as pl
from jax.experimental.pallas import tpu as pltpu

_K = 16
_GROUPS = 4
_EPS = 1e-5
_F32 = jnp.float32
_HI = jax.lax.Precision.HIGHEST


def _mm(a, b, ca, cb, precision=None):
    return jax.lax.dot_general(a, b, (((ca,), (cb,)), ((), ())),
                               preferred_element_type=_F32,
                               precision=precision)


def _fps_kernel(coor_ref, idx_ref, coorq_ref, dist_sc, far_sc, *, M):
    B = coor_ref.shape[0]
    N = coor_ref.shape[2]
    x0 = coor_ref[:, 0, :]
    x1 = coor_ref[:, 1, :]
    x2 = coor_ref[:, 2, :]
    iota_n = jax.lax.broadcasted_iota(jnp.int32, (B, N), 1)
    iota_m = jax.lax.broadcasted_iota(jnp.int32, (B, M), 1)

    dist_sc[...] = jnp.full((B, N), 1e10, _F32)
    far_sc[...] = jnp.zeros((B, 1), jnp.int32)
    idx_ref[...] = jnp.zeros((B, M), jnp.int32)
    coorq_ref[...] = jnp.zeros((B, 3, M), _F32)

    def body(i, carry):
        far = far_sc[...]
        sel = iota_m == i
        idx_ref[...] = jnp.where(sel, far, idx_ref[...])
        oh = (iota_n == far).astype(_F32)
        c0 = jnp.sum(x0 * oh, axis=1, keepdims=True)
        c1 = jnp.sum(x1 * oh, axis=1, keepdims=True)
        c2 = jnp.sum(x2 * oh, axis=1, keepdims=True)
        coorq_ref[:, 0, :] = jnp.where(sel, c0, coorq_ref[:, 0, :])
        coorq_ref[:, 1, :] = jnp.where(sel, c1, coorq_ref[:, 1, :])
        coorq_ref[:, 2, :] = jnp.where(sel, c2, coorq_ref[:, 2, :])
        d0 = x0 - c0
        d1 = x1 - c1
        d2 = x2 - c2
        d = d0 * d0 + d1 * d1 + d2 * d2
        dist = jnp.minimum(dist_sc[...], d)
        dist_sc[...] = dist
        mx = jnp.max(dist, axis=1, keepdims=True)
        far_sc[...] = jnp.min(jnp.where(dist == mx, iota_n, N), axis=1,
                              keepdims=True)
        return carry

    jax.lax.fori_loop(0, M, body, 0)


def _fps(coor, M):
    B = coor.shape[0]
    N = coor.shape[2]
    return pl.pallas_call(
        functools.partial(_fps_kernel, M=M),
        out_shape=[jax.ShapeDtypeStruct((B, M), jnp.int32),
                   jax.ShapeDtypeStruct((B, 3, M), _F32)],
        scratch_shapes=[pltpu.VMEM((B, N), _F32),
                        pltpu.VMEM((B, 1), jnp.int32)],
    )(coor)


def _group_mat(O):
    Og = O // _GROUPS
    a = jax.lax.broadcasted_iota(jnp.int32, (O, _GROUPS), 0) // Og
    b = jax.lax.broadcasted_iota(jnp.int32, (O, _GROUPS), 1)
    return (a == b).astype(_F32)


def _dist(cq, ck):
    qk = _mm(cq, ck, 0, 0)
    qsq = jnp.sum(cq * cq, axis=0, keepdims=True)
    ksq = jnp.sum(ck * ck, axis=0, keepdims=True)
    qcol = _mm(qsq, jnp.ones((1, 1), _F32), 0, 0, _HI)
    d = -2.0 * qk
    d = d + qcol
    return d + ksq


def _topk_conv(d0, fkT, fqT, W, nt, d_sc, mx_sc, s1_sc, s2_sc,
               Mx_ref, stats_ref):
    TQ, Nk = d0.shape
    O = W.shape[0]
    d_sc[...] = d0
    mx_sc[...] = jnp.full(mx_sc.shape, -1e30, _F32)
    s1_sc[...] = jnp.zeros(s1_sc.shape, _F32)
    s2_sc[...] = jnp.zeros(s2_sc.shape, _F32)
    iota = jax.lax.broadcasted_iota(jnp.int32, (TQ, Nk), 1)
    bf = jnp.bfloat16
    f_hi = fkT.astype(bf)
    r1 = fkT - f_hi.astype(_F32)
    f_mid = r1.astype(bf)
    f_lo = (r1 - f_mid.astype(_F32)).astype(bf)

    def body(j, carry):
        dd = d_sc[...]
        istar = jnp.argmin(dd, axis=1, keepdims=True)
        sel = iota == istar
        d_sc[...] = jnp.where(sel, 1e30, dd)
        oh = sel.astype(bf)
        fkj = (_mm(f_hi, oh, 1, 1) + _mm(f_mid, oh, 1, 1)
               + _mm(f_lo, oh, 1, 1))
        edge = jnp.concatenate([fkj - fqT, fqT], axis=0)
        cv = _mm(W, edge, 1, 0)
        mx_sc[...] = jnp.maximum(mx_sc[...], cv)
        s1_sc[...] = s1_sc[...] + cv
        s2_sc[...] = s2_sc[...] + cv * cv
        return carry

    jax.lax.fori_loop(0, _K, body, 0)

    Mx_ref[0] = mx_sc[...]
    Mg = _group_mat(O)
    t1 = jnp.sum(s1_sc[...], axis=1, keepdims=True)
    t2 = jnp.sum(s2_sc[...], axis=1, keepdims=True)
    gs1 = _mm(Mg, t1, 0, 0, _HI)
    gs2 = _mm(Mg, t2, 0, 0, _HI)
    st = jnp.concatenate([gs1, gs2], axis=1)

    @pl.when(nt == 0)
    def _():
        stats_ref[0] = st

    @pl.when(nt != 0)
    def _():
        stats_ref[0] = stats_ref[0] + st


def _stage1_kernel(cq_ref, ck_ref, Win_ref, b_ref, W_ref,
                   Mx_ref, stats_ref, d_sc, mx_sc, s1_sc, s2_sc):
    cq = cq_ref[0]
    ck = ck_ref[0]
    fkT = _mm(Win_ref[...], ck, 1, 0) + b_ref[...]
    fqT = _mm(Win_ref[...], cq, 1, 0) + b_ref[...]
    _topk_conv(_dist(cq, ck), fkT, fqT, W_ref[...], pl.program_id(1),
               d_sc, mx_sc, s1_sc, s2_sc, Mx_ref, stats_ref)


def _stage_self_kernel(ck_ref, fk_ref, W_ref,
                       Mx_ref, stats_ref, d_sc, mx_sc, s1_sc, s2_sc):
    ck = ck_ref[0]
    fk = fk_ref[0]
    _topk_conv(_dist(ck, ck), fk, fk, W_ref[...], 0,
               d_sc, mx_sc, s1_sc, s2_sc, Mx_ref, stats_ref)


def _stage_gather_kernel(cq_ref, ck_ref, fk_ref, idx_ref, W_ref,
                         Mx_ref, stats_ref, d_sc, mx_sc, s1_sc, s2_sc):
    cq = cq_ref[0]
    ck = ck_ref[0]
    fk = fk_ref[0]
    Nk = fk.shape[1]
    Nq = cq.shape[1]
    idx = idx_ref[0]
    PT = (jax.lax.broadcasted_iota(jnp.int32, (Nk, Nq), 0) == idx).astype(_F32)
    fqT = _mm(fk, PT, 1, 0, _HI)
    _topk_conv(_dist(cq, ck), fk, fqT, W_ref[...], 0,
               d_sc, mx_sc, s1_sc, s2_sc, Mx_ref, stats_ref)


def _finalize_kernel(Mx_ref, stats_ref, gamma_ref, beta_ref, out_ref, *,
                     count):
    st = stats_ref[0]
    mean_g = st[:, 0:1] / count
    var_g = st[:, 1:2] / count - mean_g * mean_g
    rstd_g = jax.lax.rsqrt(var_g + _EPS)
    O = out_ref.shape[1]
    Mg = _group_mat(O)
    mean_c = _mm(Mg, mean_g, 1, 0, _HI)
    rstd_c = _mm(Mg, rstd_g, 1, 0, _HI)
    y = (Mx_ref[0] - mean_c) * (rstd_c * gamma_ref[...]) + beta_ref[...]
    out_ref[0] = jnp.where(y > 0, y, 0.2 * y)


def _stage_scratch(TQ, Nk, O):
    return [pltpu.VMEM((TQ, Nk), _F32), pltpu.VMEM((O, TQ), _F32),
            pltpu.VMEM((O, TQ), _F32), pltpu.VMEM((O, TQ), _F32)]


def _full(shape):
    nd = len(shape)
    return pl.BlockSpec(shape, lambda *idx: (0,) * nd)


def _finalize(Mx, stats, gamma, beta, count):
    B, O, Nq = Mx.shape
    return pl.pallas_call(
        functools.partial(_finalize_kernel, count=count),
        grid=(B,),
        in_specs=[pl.BlockSpec((1, O, Nq), lambda b: (b, 0, 0)),
                  pl.BlockSpec((1, _GROUPS, 2), lambda b: (b, 0, 0)),
                  _full(gamma.shape), _full(beta.shape)],
        out_specs=pl.BlockSpec((1, O, Nq), lambda b: (b, 0, 0)),
        out_shape=jax.ShapeDtypeStruct((B, O, Nq), _F32),
    )(Mx, stats, gamma, beta)


def _run_stage1(x, W_in, b_in, W1, gamma, beta, TQ=512):
    B, _, N = x.shape
    O = W1.shape[0]
    NT = N // TQ
    Mx, stats = pl.pallas_call(
        _stage1_kernel,
        grid=(B, NT),
        in_specs=[pl.BlockSpec((1, 3, TQ), lambda b, t: (b, 0, t)),
                  pl.BlockSpec((1, 3, N), lambda b, t: (b, 0, 0)),
                  _full(W_in.shape), _full(b_in.shape), _full(W1.shape)],
        out_specs=[pl.BlockSpec((1, O, TQ), lambda b, t: (b, 0, t)),
                   pl.BlockSpec((1, _GROUPS, 2), lambda b, t: (b, 0, 0))],
        out_shape=[jax.ShapeDtypeStruct((B, O, N), _F32),
                   jax.ShapeDtypeStruct((B, _GROUPS, 2), _F32)],
        scratch_shapes=_stage_scratch(TQ, N, O),
    )(x, x, W_in, b_in, W1)
    return _finalize(Mx, stats, gamma, beta, float(N * _K * (O // _GROUPS)))


def _run_stage_self(coor, f, W, gamma, beta):
    B, _, N = coor.shape
    C = f.shape[1]
    O = W.shape[0]
    Mx, stats = pl.pallas_call(
        _stage_self_kernel,
        grid=(B,),
        in_specs=[pl.BlockSpec((1, 3, N), lambda b: (b, 0, 0)),
                  pl.BlockSpec((1, C, N), lambda b: (b, 0, 0)),
                  _full(W.shape)],
        out_specs=[pl.BlockSpec((1, O, N), lambda b: (b, 0, 0)),
                   pl.BlockSpec((1, _GROUPS, 2), lambda b: (b, 0, 0))],
        out_shape=[jax.ShapeDtypeStruct((B, O, N), _F32),
                   jax.ShapeDtypeStruct((B, _GROUPS, 2), _F32)],
        scratch_shapes=_stage_scratch(N, N, O),
    )(coor, f, W)
    return _finalize(Mx, stats, gamma, beta, float(N * _K * (O // _GROUPS)))


def _run_stage_gather(coor_q, coor_k, f_k, idx, W, gamma, beta):
    B, _, Nq = coor_q.shape
    Nk = coor_k.shape[2]
    C = f_k.shape[1]
    O = W.shape[0]
    idx3 = idx.reshape(B, 1, Nq)
    Mx, stats = pl.pallas_call(
        _stage_gather_kernel,
        grid=(B,),
        in_specs=[pl.BlockSpec((1, 3, Nq), lambda b: (b, 0, 0)),
                  pl.BlockSpec((1, 3, Nk), lambda b: (b, 0, 0)),
                  pl.BlockSpec((1, C, Nk), lambda b: (b, 0, 0)),
                  pl.BlockSpec((1, 1, Nq), lambda b: (b, 0, 0)),
                  _full(W.shape)],
        out_specs=[pl.BlockSpec((1, O, Nq), lambda b: (b, 0, 0)),
                   pl.BlockSpec((1, _GROUPS, 2), lambda b: (b, 0, 0))],
        out_shape=[jax.ShapeDtypeStruct((B, O, Nq), _F32),
                   jax.ShapeDtypeStruct((B, _GROUPS, 2), _F32)],
        scratch_shapes=_stage_scratch(Nq, Nk, O),
    )(coor_q, coor_k, f_k, idx3, W)
    return _finalize(Mx, stats, gamma, beta, float(Nq * _K * (O // _GROUPS)))


def kernel(x, W_in, b_in, W1, g1, bt1, W2, g2, bt2, W3, g3, bt3, W4, g4, bt4):
    bcol = b_in.reshape(-1, 1)
    col = lambda v: v.reshape(-1, 1)

    f1 = _run_stage1(x, W_in, bcol, W1, col(g1), col(bt1))
    idx1, coorq1 = _fps(x, 512)
    f2 = _run_stage_gather(coorq1, x, f1, idx1, W2, col(g2), col(bt2))
    f3 = _run_stage_self(coorq1, f2, W3, col(g3), col(bt3))
    idx2, coorq2 = _fps(coorq1, 128)
    f4 = _run_stage_gather(coorq2, coorq1, f3, idx2, W4, col(g4), col(bt4))
    return coorq2, f4


# speedup vs baseline: 2.0302x; 1.3777x over previous
import functools

import jax
import jax.numpy as jnp
from jax.experimental import pallas as pl
from jax.experimental.pallas import tpu as pltpu

_K = 16
_GROUPS = 4
_EPS = 1e-5
_F32 = jnp.float32
_HI = jax.lax.Precision.HIGHEST


def _mm(a, b, ca, cb, precision=None):
    return jax.lax.dot_general(a, b, (((ca,), (cb,)), ((), ())),
                               preferred_element_type=_F32,
                               precision=precision)


def _fps_kernel(coor_ref, idx_ref, coorq_ref, dist_sc, far_sc, *, M):
    B = coor_ref.shape[0]
    N = coor_ref.shape[2]
    x0 = coor_ref[:, 0, :]
    x1 = coor_ref[:, 1, :]
    x2 = coor_ref[:, 2, :]
    iota_n = jax.lax.broadcasted_iota(jnp.int32, (B, N), 1)
    iota_m = jax.lax.broadcasted_iota(jnp.int32, (B, M), 1)

    dist_sc[...] = jnp.full((B, N), 1e10, _F32)
    far_sc[...] = jnp.zeros((B, 1), jnp.int32)
    idx_ref[...] = jnp.zeros((B, M), jnp.int32)
    coorq_ref[...] = jnp.zeros((B, 3, M), _F32)

    def body(i, carry):
        far = far_sc[...]
        sel = iota_m == i
        idx_ref[...] = jnp.where(sel, far, idx_ref[...])
        oh = (iota_n == far).astype(_F32)
        c0 = jnp.sum(x0 * oh, axis=1, keepdims=True)
        c1 = jnp.sum(x1 * oh, axis=1, keepdims=True)
        c2 = jnp.sum(x2 * oh, axis=1, keepdims=True)
        coorq_ref[:, 0, :] = jnp.where(sel, c0, coorq_ref[:, 0, :])
        coorq_ref[:, 1, :] = jnp.where(sel, c1, coorq_ref[:, 1, :])
        coorq_ref[:, 2, :] = jnp.where(sel, c2, coorq_ref[:, 2, :])
        d0 = x0 - c0
        d1 = x1 - c1
        d2 = x2 - c2
        d = d0 * d0 + d1 * d1 + d2 * d2
        dist = jnp.minimum(dist_sc[...], d)
        dist_sc[...] = dist
        mx = jnp.max(dist, axis=1, keepdims=True)
        far_sc[...] = jnp.min(jnp.where(dist == mx, iota_n, N), axis=1,
                              keepdims=True)
        return carry

    jax.lax.fori_loop(0, M, body, 0)


def _fps(coor, M):
    B = coor.shape[0]
    N = coor.shape[2]
    return pl.pallas_call(
        functools.partial(_fps_kernel, M=M),
        out_shape=[jax.ShapeDtypeStruct((B, M), jnp.int32),
                   jax.ShapeDtypeStruct((B, 3, M), _F32)],
        scratch_shapes=[pltpu.VMEM((B, N), _F32),
                        pltpu.VMEM((B, 1), jnp.int32)],
    )(coor)


def _group_mat(O):
    Og = O // _GROUPS
    a = jax.lax.broadcasted_iota(jnp.int32, (O, _GROUPS), 0) // Og
    b = jax.lax.broadcasted_iota(jnp.int32, (O, _GROUPS), 1)
    return (a == b).astype(_F32)


def _dist(cq, ck):
    qk = _mm(cq, ck, 0, 0)
    qsq = jnp.sum(cq * cq, axis=0, keepdims=True)
    ksq = jnp.sum(ck * ck, axis=0, keepdims=True)
    qcol = _mm(qsq, jnp.ones((1, 1), _F32), 0, 0, _HI)
    d = -2.0 * qk
    d = d + qcol
    return d + ksq


def _topk_conv(d0, fkT, fqT, W, nt, d_sc, mx_sc, s1_sc, s2_sc,
               Mx_ref, stats_ref):
    TQ, Nk = d0.shape
    O = W.shape[0]
    d_sc[...] = d0
    mx_sc[...] = jnp.full(mx_sc.shape, -1e30, _F32)
    s1_sc[...] = jnp.zeros(s1_sc.shape, _F32)
    s2_sc[...] = jnp.zeros(s2_sc.shape, _F32)
    iota = jax.lax.broadcasted_iota(jnp.int32, (TQ, Nk), 1)
    bf = jnp.bfloat16
    C = fkT.shape[0]
    f_hi = fkT.astype(bf)
    r1 = fkT - f_hi.astype(_F32)
    f_mid = r1.astype(bf)
    f_lo = (r1 - f_mid.astype(_F32)).astype(bf)
    f_split = jnp.concatenate([f_hi, f_mid, f_lo], axis=0)

    def body(j, carry):
        dd = d_sc[...]
        istar = jnp.argmin(dd, axis=1, keepdims=True)
        sel = iota == istar
        d_sc[...] = jnp.where(sel, 1e30, dd)
        oh = sel.astype(bf)
        g3 = _mm(f_split, oh, 1, 1)
        fkj = (g3[:C] + g3[C:2 * C]) + g3[2 * C:]
        edge = jnp.concatenate([fkj - fqT, fqT], axis=0)
        cv = _mm(W, edge, 1, 0)
        mx_sc[...] = jnp.maximum(mx_sc[...], cv)
        s1_sc[...] = s1_sc[...] + cv
        s2_sc[...] = s2_sc[...] + cv * cv
        return carry

    jax.lax.fori_loop(0, _K, body, 0)

    Mx_ref[0] = mx_sc[...]
    Mg = _group_mat(O)
    t1 = jnp.sum(s1_sc[...], axis=1, keepdims=True)
    t2 = jnp.sum(s2_sc[...], axis=1, keepdims=True)
    gs1 = _mm(Mg, t1, 0, 0, _HI)
    gs2 = _mm(Mg, t2, 0, 0, _HI)
    st = jnp.concatenate([gs1, gs2], axis=1)

    @pl.when(nt == 0)
    def _():
        stats_ref[0] = st

    @pl.when(nt != 0)
    def _():
        stats_ref[0] = stats_ref[0] + st


def _stage1_kernel(cq_ref, ck_ref, Win_ref, b_ref, W_ref,
                   Mx_ref, stats_ref, d_sc, mx_sc, s1_sc, s2_sc):
    cq = cq_ref[0]
    ck = ck_ref[0]
    fkT = _mm(Win_ref[...], ck, 1, 0) + b_ref[...]
    fqT = _mm(Win_ref[...], cq, 1, 0) + b_ref[...]
    _topk_conv(_dist(cq, ck), fkT, fqT, W_ref[...], pl.program_id(1),
               d_sc, mx_sc, s1_sc, s2_sc, Mx_ref, stats_ref)


def _stage_self_kernel(ck_ref, fk_ref, W_ref,
                       Mx_ref, stats_ref, d_sc, mx_sc, s1_sc, s2_sc):
    ck = ck_ref[0]
    fk = fk_ref[0]
    _topk_conv(_dist(ck, ck), fk, fk, W_ref[...], 0,
               d_sc, mx_sc, s1_sc, s2_sc, Mx_ref, stats_ref)


def _stage_gather_kernel(cq_ref, ck_ref, fk_ref, idx_ref, W_ref,
                         Mx_ref, stats_ref, d_sc, mx_sc, s1_sc, s2_sc):
    cq = cq_ref[0]
    ck = ck_ref[0]
    fk = fk_ref[0]
    Nk = fk.shape[1]
    Nq = cq.shape[1]
    idx = idx_ref[0]
    PT = (jax.lax.broadcasted_iota(jnp.int32, (Nk, Nq), 0) == idx).astype(_F32)
    fqT = _mm(fk, PT, 1, 0, _HI)
    _topk_conv(_dist(cq, ck), fk, fqT, W_ref[...], 0,
               d_sc, mx_sc, s1_sc, s2_sc, Mx_ref, stats_ref)


def _finalize_kernel(Mx_ref, stats_ref, gamma_ref, beta_ref, out_ref, *,
                     count):
    st = stats_ref[0]
    mean_g = st[:, 0:1] / count
    var_g = st[:, 1:2] / count - mean_g * mean_g
    rstd_g = jax.lax.rsqrt(var_g + _EPS)
    O = out_ref.shape[1]
    Mg = _group_mat(O)
    mean_c = _mm(Mg, mean_g, 1, 0, _HI)
    rstd_c = _mm(Mg, rstd_g, 1, 0, _HI)
    y = (Mx_ref[0] - mean_c) * (rstd_c * gamma_ref[...]) + beta_ref[...]
    out_ref[0] = jnp.where(y > 0, y, 0.2 * y)


def _stage_scratch(TQ, Nk, O):
    return [pltpu.VMEM((TQ, Nk), _F32), pltpu.VMEM((O, TQ), _F32),
            pltpu.VMEM((O, TQ), _F32), pltpu.VMEM((O, TQ), _F32)]


def _full(shape):
    nd = len(shape)
    return pl.BlockSpec(shape, lambda *idx: (0,) * nd)


def _finalize(Mx, stats, gamma, beta, count):
    B, O, Nq = Mx.shape
    return pl.pallas_call(
        functools.partial(_finalize_kernel, count=count),
        grid=(B,),
        in_specs=[pl.BlockSpec((1, O, Nq), lambda b: (b, 0, 0)),
                  pl.BlockSpec((1, _GROUPS, 2), lambda b: (b, 0, 0)),
                  _full(gamma.shape), _full(beta.shape)],
        out_specs=pl.BlockSpec((1, O, Nq), lambda b: (b, 0, 0)),
        out_shape=jax.ShapeDtypeStruct((B, O, Nq), _F32),
    )(Mx, stats, gamma, beta)


def _run_stage1(x, W_in, b_in, W1, gamma, beta, TQ=512):
    B, _, N = x.shape
    O = W1.shape[0]
    NT = N // TQ
    Mx, stats = pl.pallas_call(
        _stage1_kernel,
        grid=(B, NT),
        in_specs=[pl.BlockSpec((1, 3, TQ), lambda b, t: (b, 0, t)),
                  pl.BlockSpec((1, 3, N), lambda b, t: (b, 0, 0)),
                  _full(W_in.shape), _full(b_in.shape), _full(W1.shape)],
        out_specs=[pl.BlockSpec((1, O, TQ), lambda b, t: (b, 0, t)),
                   pl.BlockSpec((1, _GROUPS, 2), lambda b, t: (b, 0, 0))],
        out_shape=[jax.ShapeDtypeStruct((B, O, N), _F32),
                   jax.ShapeDtypeStruct((B, _GROUPS, 2), _F32)],
        scratch_shapes=_stage_scratch(TQ, N, O),
    )(x, x, W_in, b_in, W1)
    return _finalize(Mx, stats, gamma, beta, float(N * _K * (O // _GROUPS)))


def _run_stage_self(coor, f, W, gamma, beta):
    B, _, N = coor.shape
    C = f.shape[1]
    O = W.shape[0]
    Mx, stats = pl.pallas_call(
        _stage_self_kernel,
        grid=(B,),
        in_specs=[pl.BlockSpec((1, 3, N), lambda b: (b, 0, 0)),
                  pl.BlockSpec((1, C, N), lambda b: (b, 0, 0)),
                  _full(W.shape)],
        out_specs=[pl.BlockSpec((1, O, N), lambda b: (b, 0, 0)),
                   pl.BlockSpec((1, _GROUPS, 2), lambda b: (b, 0, 0))],
        out_shape=[jax.ShapeDtypeStruct((B, O, N), _F32),
                   jax.ShapeDtypeStruct((B, _GROUPS, 2), _F32)],
        scratch_shapes=_stage_scratch(N, N, O),
    )(coor, f, W)
    return _finalize(Mx, stats, gamma, beta, float(N * _K * (O // _GROUPS)))


def _run_stage_gather(coor_q, coor_k, f_k, idx, W, gamma, beta):
    B, _, Nq = coor_q.shape
    Nk = coor_k.shape[2]
    C = f_k.shape[1]
    O = W.shape[0]
    idx3 = idx.reshape(B, 1, Nq)
    Mx, stats = pl.pallas_call(
        _stage_gather_kernel,
        grid=(B,),
        in_specs=[pl.BlockSpec((1, 3, Nq), lambda b: (b, 0, 0)),
                  pl.BlockSpec((1, 3, Nk), lambda b: (b, 0, 0)),
                  pl.BlockSpec((1, C, Nk), lambda b: (b, 0, 0)),
                  pl.BlockSpec((1, 1, Nq), lambda b: (b, 0, 0)),
                  _full(W.shape)],
        out_specs=[pl.BlockSpec((1, O, Nq), lambda b: (b, 0, 0)),
                   pl.BlockSpec((1, _GROUPS, 2), lambda b: (b, 0, 0))],
        out_shape=[jax.ShapeDtypeStruct((B, O, Nq), _F32),
                   jax.ShapeDtypeStruct((B, _GROUPS, 2), _F32)],
        scratch_shapes=_stage_scratch(Nq, Nk, O),
    )(coor_q, coor_k, f_k, idx3, W)
    return _finalize(Mx, stats, gamma, beta, float(Nq * _K * (O // _GROUPS)))


def kernel(x, W_in, b_in, W1, g1, bt1, W2, g2, bt2, W3, g3, bt3, W4, g4, bt4):
    bcol = b_in.reshape(-1, 1)
    col = lambda v: v.reshape(-1, 1)

    f1 = _run_stage1(x, W_in, bcol, W1, col(g1), col(bt1))
    idx1, coorq1 = _fps(x, 512)
    f2 = _run_stage_gather(coorq1, x, f1, idx1, W2, col(g2), col(bt2))
    f3 = _run_stage_self(coorq1, f2, W3, col(g3), col(bt3))
    idx2, coorq2 = _fps(coorq1, 128)
    f4 = _run_stage_gather(coorq2, coorq1, f3, idx2, W4, col(g4), col(bt4))
    return coorq2, f4
